# baseline (speedup 1.0000x reference)
import functools

import jax
import jax.numpy as jnp
from jax import lax
from jax.experimental import pallas as pl
from jax.experimental.pallas import tpu as pltpu
from jax.experimental.pallas import tpu_sc as plsc

B = 16384
D = 16
_NC = 2
_NS = 16
_NW = _NC * _NS
_BPW = B // _NW
_CHUNK = 128
_NCHUNK = _BPW // _CHUNK

_mesh = plsc.VectorSubcoreMesh(core_axis_name="c", subcore_axis_name="s")


@functools.partial(
    pl.kernel,
    mesh=_mesh,
    compiler_params=pltpu.CompilerParams(use_tc_tiling_on_sc=False),
    out_type=[jax.ShapeDtypeStruct((B, D), jnp.float32)] * 4,
    scratch_types=[
        pltpu.VMEM((_NCHUNK, _CHUNK), jnp.int32),
        pltpu.VMEM((_NCHUNK, _CHUNK), jnp.int32),
        pltpu.VMEM((_BPW, D), jnp.float32),
        pltpu.VMEM((_BPW, D), jnp.float32),
        pltpu.VMEM((_BPW, D), jnp.float32),
        pltpu.VMEM((_BPW, D), jnp.float32),
        pltpu.SemaphoreType.DMA,
    ],
)
def _sc_gather(uidx_hbm, iidx_hbm, t_umf, t_imf, t_umlp, t_imlp,
               o_umf, o_imf, o_umlp, o_imlp,
               uidx_v, iidx_v, r_umf, r_imf, r_umlp, r_imlp, sem):
    wid = lax.axis_index("s") * _NC + lax.axis_index("c")
    base = wid * _BPW
    crow = wid * _NCHUNK
    pltpu.sync_copy(uidx_hbm.at[pl.ds(crow, _NCHUNK)], uidx_v)
    pltpu.sync_copy(iidx_hbm.at[pl.ds(crow, _NCHUNK)], iidx_v)
    copies = []
    for j in range(_NCHUNK):
        sl = pl.ds(j * _CHUNK, _CHUNK)
        copies.append(pltpu.async_copy(t_umf.at[uidx_v.at[j]], r_umf.at[sl], sem))
        copies.append(pltpu.async_copy(t_imf.at[iidx_v.at[j]], r_imf.at[sl], sem))
        copies.append(pltpu.async_copy(t_umlp.at[uidx_v.at[j]], r_umlp.at[sl], sem))
        copies.append(pltpu.async_copy(t_imlp.at[iidx_v.at[j]], r_imlp.at[sl], sem))
    for c in copies:
        c.wait()
    out_sl = pl.ds(base, _BPW)
    pltpu.sync_copy(r_umf, o_umf.at[out_sl])
    pltpu.sync_copy(r_imf, o_imf.at[out_sl])
    pltpu.sync_copy(r_umlp, o_umlp.at[out_sl])
    pltpu.sync_copy(r_imlp, o_imlp.at[out_sl])


_BLK = 2048


def _mlp_body(umf, imf, umlp, imlp, w1a, w1b, b1, w2, b2, woa, wob, bo, out):
    pred = umf[...] * imf[...]
    h = jnp.dot(umlp[...], w1a[...], preferred_element_type=jnp.float32)
    h = h + jnp.dot(imlp[...], w1b[...], preferred_element_type=jnp.float32)
    h = jnp.maximum(h + b1[...], 0.0)
    h = jnp.dot(h, w2[...], preferred_element_type=jnp.float32) + b2[...]
    h = jnp.maximum(h, 0.0)
    r = jnp.dot(pred, woa[...], preferred_element_type=jnp.float32)
    r = r + jnp.dot(h, wob[...], preferred_element_type=jnp.float32)
    out[...] = r + bo[...]


def _mlp(umf, imf, umlp, imlp, w1a, w1b, b1, w2, b2, woa, wob, bo):
    row = pl.BlockSpec((_BLK, D), lambda i: (i, 0))

    def full(a):
        return pl.BlockSpec(a.shape, lambda i: (0,) * a.ndim)

    return pl.pallas_call(
        _mlp_body,
        grid=(B // _BLK,),
        in_specs=[row, row, row, row,
                  full(w1a), full(w1b), full(b1), full(w2), full(b2),
                  full(woa), full(wob), full(bo)],
        out_specs=pl.BlockSpec((_BLK, 1), lambda i: (i, 0)),
        out_shape=jax.ShapeDtypeStruct((B, 1), jnp.float32),
    )(umf, imf, umlp, imlp, w1a, w1b, b1, w2, b2, woa, wob, bo)


def kernel(user_indices, item_indices, emb_user_mf, emb_item_mf,
           emb_user_mlp, emb_item_mlp, W1, b1, W2, b2, Wout, bout):
    uidx = user_indices.astype(jnp.int32).reshape(B // _CHUNK, _CHUNK)
    iidx = item_indices.astype(jnp.int32).reshape(B // _CHUNK, _CHUNK)
    umf, imf, umlp, imlp = _sc_gather(
        uidx, iidx, emb_user_mf, emb_item_mf, emb_user_mlp, emb_item_mlp)
    w1a = W1[:, :D].T
    w1b = W1[:, D:].T
    w2t = W2.T
    woa = Wout[:, :D].T
    wob = Wout[:, D:].T
    return _mlp(umf, imf, umlp, imlp, w1a, w1b, b1.reshape(1, -1), w2t,
                b2.reshape(1, -1), woa, wob, bout.reshape(1, 1))
